# baseline (speedup 1.0000x reference)
import jax
import jax.numpy as jnp
from jax import lax
from jax.experimental import pallas as pl
from jax.experimental.pallas import tpu as pltpu
from jax.experimental.pallas import tpu_sc as plsc

B, L, T, V = 1024, 26, 50, 1001
TPAD = 56
VPAD = 1008
NC, NS = 2, 16
NB = 32
NQ = NB // 16
NV = TPAD * NQ
JP = L // 2
TPS = 4
SPS = NS // TPS
MAGIC = 0x5F3759DF


def _process_unit(tok_v, w_v, hist, tsv, cols, cb):
    def _in(t, q):
        return pl.ds(pl.multiple_of(cb + 16 * q, 16), 16)

    for k in range(NV):
        t, q = k // NQ, k % NQ
        tok = tok_v[t, _in(t, q)]
        tsv[pl.ds(16 * k, 16)] = tok
        plsc.addupdate_scatter(hist, [tok, cols[q]], w_v[t, _in(t, q)])
    part = [jnp.zeros((16,), jnp.float32) for _ in range(NQ)]
    cs = []
    for k in range(NV):
        t, q = k // NQ, k % NQ
        tok = tok_v[t, _in(t, q)]
        c = plsc.load_gather(hist, [tok, cols[q]])
        cs.append(c)
        part[q] = part[q] + c * w_v[t, _in(t, q)]
    ys = []
    for q in range(NQ):
        xv = jnp.maximum(part[q], 1e-12)
        iv = jnp.int32(MAGIC) - lax.shift_right_logical(
            plsc.bitcast(xv, jnp.int32), 1)
        y = plsc.bitcast(iv, jnp.float32)
        for _ in range(3):
            y = y * (1.5 - 0.5 * xv * y * y)
        ys.append(y)
    for k in range(NV):
        t, q = k // NQ, k % NQ
        tok = tok_v[t, _in(t, q)]
        plsc.store_scatter(hist, [tok, cols[q]], cs[k] * ys[q])


def _rezero(hist, tsv, cols):
    zv = jnp.zeros((16,), jnp.float32)
    for k in range(NV):
        tok = tsv[pl.ds(16 * k, 16)]
        plsc.store_scatter(hist, [tok, cols[k % NQ]], zv)


def _sc_body(tok_hbm, w_hbm, out_hbm,
             tok0, tok1, w0, w1, hist, tsv, slabs,
             osem, tsem0, tsem1, wsem0, wsem1):
    sc = lax.axis_index("c")
    tile = lax.axis_index("s")
    slab = tile // TPS
    cb = (tile % TPS) * NB
    bt = pl.multiple_of((sc * SPS + slab) * 128, 128)
    is_issuer = tile % TPS == 0
    ins = ((tok0, w0, tsem0, wsem0), (tok1, w1, tsem1, wsem1))
    lanes = lax.iota(jnp.int32, 16)
    cols = tuple(lanes + 16 * q for q in range(NQ))

    def zero_body(i, carry):
        zv = jnp.zeros((16,), jnp.float32)
        for q in range(NQ):
            hist[i, pl.ds(16 * q, 16)] = zv
        return carry

    lax.fori_loop(0, V, zero_body, 0)

    for p in range(2):
        tok_v, w_v, tsem, wsem = ins[p]
        pltpu.async_copy(tok_hbm.at[p, :, pl.ds(bt, 128)], tok_v, tsem)
        pltpu.async_copy(w_hbm.at[p, :, pl.ds(bt, 128)], w_v, wsem)

    def pair_body(j, carry):
        for p in range(2):
            tok_v, w_v, tsem, wsem = ins[p]
            fl = j * 2 + p
            pltpu.make_async_copy(
                tok_hbm.at[fl, :, pl.ds(bt, 128)], tok_v, tsem).wait()
            pltpu.make_async_copy(
                w_hbm.at[fl, :, pl.ds(bt, 128)], w_v, wsem).wait()

            _process_unit(tok_v, w_v, hist, tsv, cols, cb)

            @pl.when(j < JP - 1)
            def _prefetch():
                pltpu.async_copy(
                    tok_hbm.at[fl + 2, :, pl.ds(bt, 128)], tok_v, tsem)
                pltpu.async_copy(
                    w_hbm.at[fl + 2, :, pl.ds(bt, 128)], w_v, wsem)

            @pl.when((fl > 0) & is_issuer)
            def _wait_slab():
                pltpu.make_async_copy(
                    slabs.at[slab],
                    out_hbm.at[fl - 1, :, pl.ds(bt, 128)], osem).wait()

            plsc.subcore_barrier()
            pltpu.sync_copy(
                hist, slabs.at[slab, pl.ds(0, V), pl.ds(cb, NB)])
            _rezero(hist, tsv, cols)
            plsc.subcore_barrier()

            @pl.when(is_issuer)
            def _fire_slab():
                pltpu.async_copy(slabs.at[slab],
                                 out_hbm.at[fl, :, pl.ds(bt, 128)], osem)
        return carry

    lax.fori_loop(0, JP, pair_body, 0)

    @pl.when(is_issuer)
    def _drain():
        pltpu.make_async_copy(
            slabs.at[slab],
            out_hbm.at[L - 1, :, pl.ds(bt, 128)], osem).wait()


@jax.jit
def kernel(token_ids, weights):
    tok_t = jnp.transpose(token_ids, (1, 2, 0))
    w_t = jnp.transpose(weights, (1, 2, 0))

    mesh = plsc.VectorSubcoreMesh(
        core_axis_name="c", subcore_axis_name="s", num_cores=NC,
        num_subcores=NS)
    tok_p = jnp.concatenate(
        [tok_t, jnp.zeros((L, TPAD - T, B), jnp.int32)], axis=1)
    w_p = jnp.concatenate(
        [w_t, jnp.zeros((L, TPAD - T, B), jnp.float32)], axis=1)
    out_t = pl.kernel(
        _sc_body,
        out_type=jax.ShapeDtypeStruct((L, VPAD, B), jnp.float32),
        mesh=mesh,
        compiler_params=pltpu.CompilerParams(
            needs_layout_passes=False, use_tc_tiling_on_sc=False),
        scratch_types=[
            pltpu.VMEM((TPAD, 128), jnp.int32),
            pltpu.VMEM((TPAD, 128), jnp.int32),
            pltpu.VMEM((TPAD, 128), jnp.float32),
            pltpu.VMEM((TPAD, 128), jnp.float32),
            pltpu.VMEM((V, NB), jnp.float32),
            pltpu.VMEM((NV * 16,), jnp.int32),
            pltpu.VMEM_SHARED((SPS, VPAD, 128), jnp.float32),
            pltpu.SemaphoreType.DMA,
            pltpu.SemaphoreType.DMA,
            pltpu.SemaphoreType.DMA,
            pltpu.SemaphoreType.DMA,
            pltpu.SemaphoreType.DMA,
        ],
    )(tok_p, w_p)
    return jnp.transpose(out_t[:, :V, :], (2, 0, 1))


# speedup vs baseline: 1.1723x; 1.1723x over previous
import jax
import jax.numpy as jnp
from jax import lax
from jax.experimental import pallas as pl
from jax.experimental.pallas import tpu as pltpu
from jax.experimental.pallas import tpu_sc as plsc

B, L, T, V = 1024, 26, 50, 1001
LT_FULL = L // 8
LTAIL = L - 8 * LT_FULL
NC, NS = 2, 16
WORKERS = NC * NS
BPW = B // WORKERS
JP = BPW // 2
VPR = (T + 15) // 16
MAGIC = 0x5F3759DF

UNIT_ROWS = (8, 8, 8, LTAIL)
UNIT_L0 = (0, 8, 16, 24)


def _row_vecs(tok_v, w_v, l, lanes):
    lv = jnp.full((16,), l, jnp.int32)
    toks, ws = [], []
    for m in range(VPR):
        cols = jnp.minimum(m * 16 + lanes, T - 1)
        tok = plsc.load_gather(tok_v, [lv, cols])
        w = plsc.load_gather(w_v, [lv, cols])
        if m == VPR - 1:
            w = jnp.where(m * 16 + lanes < T, w, 0.0)
        toks.append(tok)
        ws.append(w)
    return toks, ws


def _process_unit(tok_v, w_v, buf, tsave, lt, lanes):
    nrows = UNIT_ROWS[lt]
    rows = []
    for r in range(nrows):
        toks, ws = _row_vecs(tok_v, w_v, UNIT_L0[lt] + r, lanes)
        rv = jnp.full((16,), r, jnp.int32)
        rows.append((rv, toks, ws))
        for m in range(VPR):
            tsave[pl.ds((r * VPR + m) * 16, 16)] = toks[m]
            plsc.addupdate_scatter(buf, [rv, toks[m]], ws[m])
    for r in range(nrows):
        rv, toks, ws = rows[r]
        cs = [plsc.load_gather(buf, [rv, toks[m]]) for m in range(VPR)]
        part = jnp.zeros((16,), jnp.float32)
        for m in range(VPR):
            part = part + cs[m] * ws[m]
        s = jnp.maximum(jnp.sum(part), 1e-12)
        xv = jnp.broadcast_to(s, (16,))
        iv = jnp.int32(MAGIC) - lax.shift_right_logical(
            plsc.bitcast(xv, jnp.int32), 1)
        y = plsc.bitcast(iv, jnp.float32)
        for _ in range(3):
            y = y * (1.5 - 0.5 * xv * y * y)
        for m in range(VPR):
            plsc.store_scatter(buf, [rv, toks[m]], cs[m] * y)


def _rezero(buf, tsave, nrows):
    zv = jnp.zeros((16,), jnp.float32)
    for r in range(nrows):
        rv = jnp.full((16,), r, jnp.int32)
        for m in range(VPR):
            tok = tsave[pl.ds((r * VPR + m) * 16, 16)]
            plsc.store_scatter(buf, [rv, tok], zv)


def _out_slice(out_hbm, b, lt):
    return out_hbm.at[b, pl.ds(UNIT_L0[lt], UNIT_ROWS[lt]), :]


def _buf_slice(buf, lt):
    return buf.at[pl.ds(0, UNIT_ROWS[lt])] if UNIT_ROWS[lt] != 8 else buf


def _sc_body(tok_hbm, w_hbm, out_hbm,
             tok0, tok1, w0, w1, buf0, buf1, tsv0, tsv1,
             osem0, osem1, tsem0, tsem1, wsem0, wsem1):
    wid = lax.axis_index("s") * NC + lax.axis_index("c")
    b0 = wid * BPW
    bufs = (buf0, buf1)
    tsvs = (tsv0, tsv1)
    osems = (osem0, osem1)
    ins = ((tok0, w0, tsem0, wsem0), (tok1, w1, tsem1, wsem1))

    lanes = lax.iota(jnp.int32, 16)

    def zero_body(i, carry):
        r = jnp.broadcast_to(i // 63, (16,))
        c = (i % 63) * 16 + lanes
        zv = jnp.zeros((16,), jnp.float32)
        plsc.store_scatter(buf0, [r, c], zv, mask=c < V)
        plsc.store_scatter(buf1, [r, c], zv, mask=c < V)
        return carry

    lax.fori_loop(0, 8 * 63, zero_body, 0)

    for q in range(2):
        tok_v, w_v, tsem, wsem = ins[q]
        b = b0 + q
        pltpu.async_copy(tok_hbm.at[b], tok_v, tsem)
        pltpu.async_copy(w_hbm.at[b], w_v, wsem)

    def pair_body(j, carry):
        for q in range(2):
            tok_v, w_v, tsem, wsem = ins[q]
            b = b0 + j * 2 + q
            pltpu.make_async_copy(tok_hbm.at[b], tok_v, tsem).wait()
            pltpu.make_async_copy(w_hbm.at[b], w_v, wsem).wait()

            for lt in range(4):
                p = lt % 2
                buf, tsv, osem = bufs[p], tsvs[p], osems[p]
                prev_lt = lt - 2 if lt >= 2 else lt + 2
                prev_b = b if lt >= 2 else b - 1

                def _reclaim(prev_b=prev_b, prev_lt=prev_lt, buf=buf,
                             tsv=tsv, osem=osem):
                    pltpu.make_async_copy(
                        _buf_slice(buf, prev_lt),
                        _out_slice(out_hbm, prev_b, prev_lt), osem).wait()
                    _rezero(buf, tsv, UNIT_ROWS[prev_lt])

                if lt >= 2 or q == 1:
                    _reclaim()
                else:
                    pl.when(j > 0)(_reclaim)

                _process_unit(tok_v, w_v, buf, tsv, lt, lanes)
                if lt == 3:
                    @pl.when(j < JP - 1)
                    def _prefetch():
                        pltpu.async_copy(tok_hbm.at[b + 2], tok_v, tsem)
                        pltpu.async_copy(w_hbm.at[b + 2], w_v, wsem)
                pltpu.async_copy(_buf_slice(buf, lt),
                                 _out_slice(out_hbm, b, lt), osem)
        return carry

    lax.fori_loop(0, JP, pair_body, 0)

    blast = b0 + BPW - 1
    for lt in (2, 3):
        p = lt % 2
        pltpu.make_async_copy(_buf_slice(bufs[p], lt),
                              _out_slice(out_hbm, blast, lt), osems[p]).wait()


@jax.jit
def kernel(token_ids, weights):
    mesh = plsc.VectorSubcoreMesh(
        core_axis_name="c", subcore_axis_name="s", num_cores=NC,
        num_subcores=NS)
    return pl.kernel(
        _sc_body,
        out_type=jax.ShapeDtypeStruct((B, L, V), jnp.float32),
        mesh=mesh,
        compiler_params=pltpu.CompilerParams(needs_layout_passes=False),
        scratch_types=[
            pltpu.VMEM((L, T), jnp.int32),
            pltpu.VMEM((L, T), jnp.int32),
            pltpu.VMEM((L, T), jnp.float32),
            pltpu.VMEM((L, T), jnp.float32),
            pltpu.VMEM((8, V), jnp.float32),
            pltpu.VMEM((8, V), jnp.float32),
            pltpu.VMEM((8 * VPR * 16,), jnp.int32),
            pltpu.VMEM((8 * VPR * 16,), jnp.int32),
            pltpu.SemaphoreType.DMA,
            pltpu.SemaphoreType.DMA,
            pltpu.SemaphoreType.DMA,
            pltpu.SemaphoreType.DMA,
            pltpu.SemaphoreType.DMA,
            pltpu.SemaphoreType.DMA,
        ],
    )(token_ids, weights)
